# baseline (speedup 1.0000x reference)
import functools
import math

import jax
import jax.numpy as jnp
from jax.experimental import pallas as pl
from jax.experimental.pallas import tpu as pltpu

_BLK = 2048


def _body(o_ref, scratch, *, blk, dim, log_base):
    i = pl.program_id(0)
    j = pl.program_id(1)

    @pl.when(j == 0)
    def _compute():
        half = dim // 2
        rows = jax.lax.broadcasted_iota(jnp.int32, (blk, half), 0).astype(jnp.float32)
        cols = jax.lax.broadcasted_iota(jnp.int32, (blk, half), 1).astype(jnp.float32)
        pos = rows + (i * blk).astype(jnp.float32)
        invden = jnp.exp(cols * jnp.float32(-2.0 * log_base / dim))
        arg = pos * invden
        scratch[:, :half] = jnp.sin(arg)
        scratch[:, half:] = jnp.cos(arg)

    o_ref[...] = scratch[...][None]


def kernel(input_tensor, weight):
    bsz, seq_len, dim = input_tensor.shape
    body = functools.partial(_body, blk=_BLK, dim=dim, log_base=math.log(10000.0))
    return pl.pallas_call(
        body,
        grid=(seq_len // _BLK, bsz),
        in_specs=[],
        out_specs=pl.BlockSpec((1, _BLK, dim), lambda i, j: (j, i, 0)),
        out_shape=jax.ShapeDtypeStruct((bsz, seq_len, dim), weight.dtype),
        scratch_shapes=[pltpu.VMEM((_BLK, dim), jnp.float32)],
    )()


# speedup vs baseline: 2.1669x; 2.1669x over previous
import functools
import math

import jax
import jax.numpy as jnp
from jax.experimental import pallas as pl
from jax.experimental.pallas import tpu as pltpu

_BLK = 2048
_SEED = 256


def _body(o_ref, scratch, coef, *, blk, dim, log_base):
    i = pl.program_id(0)
    j = pl.program_id(1)
    half = dim // 2

    @pl.when((i == 0) & (j == 0))
    def _seed():
        cols = jax.lax.broadcasted_iota(jnp.int32, (1, half), 1).astype(jnp.float32)
        invden = jnp.exp(cols * jnp.float32(-2.0 * log_base / dim))
        coef[0:1, :] = jnp.cos(blk * invden)
        coef[1:2, :] = jnp.sin(blk * invden)
        rows = jax.lax.broadcasted_iota(jnp.int32, (_SEED, half), 0).astype(jnp.float32)
        arg = rows * invden
        scratch[:_SEED, :half] = jnp.sin(arg)
        scratch[:_SEED, half:] = jnp.cos(arg)
        k = _SEED
        while k < blk:
            s0 = scratch[:k, :half]
            c0 = scratch[:k, half:]
            ca = jnp.cos(k * invden)
            sa = jnp.sin(k * invden)
            scratch[k:2 * k, :half] = s0 * ca + c0 * sa
            scratch[k:2 * k, half:] = c0 * ca - s0 * sa
            k *= 2

    @pl.when((i > 0) & (j == 0))
    def _rotate():
        s0 = scratch[:, :half]
        c0 = scratch[:, half:]
        ca = coef[0:1, :]
        sa = coef[1:2, :]
        scratch[:, :half] = s0 * ca + c0 * sa
        scratch[:, half:] = c0 * ca - s0 * sa

    o_ref[...] = scratch[...][None]


def kernel(input_tensor, weight):
    bsz, seq_len, dim = input_tensor.shape
    body = functools.partial(_body, blk=_BLK, dim=dim, log_base=math.log(10000.0))
    return pl.pallas_call(
        body,
        grid=(seq_len // _BLK, bsz),
        in_specs=[],
        out_specs=pl.BlockSpec((1, _BLK, dim), lambda i, j: (j, i, 0)),
        out_shape=jax.ShapeDtypeStruct((bsz, seq_len, dim), weight.dtype),
        scratch_shapes=[
            pltpu.VMEM((_BLK, dim), jnp.float32),
            pltpu.VMEM((2, dim // 2), jnp.float32),
        ],
    )()
